# baseline (speedup 1.0000x reference)
import functools

import jax
import jax.numpy as jnp
from jax import lax
from jax.experimental import pallas as pl
from jax.experimental.pallas import tpu as pltpu
from jax.experimental.pallas import tpu_sc as plsc

B = 16384
D = 32

_NC = 2
_NS = 16
_NW = _NC * _NS
_BPW = B // _NW
_CH = 128
_NCH = _BPW // _CH
_NROW = B // _CH


def _sc_gather3(uid2, iid2, pid2, user_table, item_table, pub_table):
    mesh = plsc.VectorSubcoreMesh(core_axis_name="c", subcore_axis_name="s")
    row_t = jax.ShapeDtypeStruct((_NROW, _CH, D), jnp.float32)

    @functools.partial(
        pl.kernel,
        out_type=[row_t, row_t, row_t],
        mesh=mesh,
        compiler_params=pltpu.CompilerParams(use_tc_tiling_on_sc=False),
        scratch_types=[
            pltpu.VMEM((_NCH, _CH), jnp.int32),
            pltpu.VMEM((_NCH, _CH), jnp.int32),
            pltpu.VMEM((_NCH, _CH), jnp.int32),
            pltpu.VMEM((_NCH, _CH, D), jnp.float32),
            pltpu.VMEM((_NCH, _CH, D), jnp.float32),
            pltpu.VMEM((_NCH, _CH, D), jnp.float32),
            pltpu.SemaphoreType.DMA,
        ],
    )
    def k(uid_h, iid_h, pid_h, ut_h, it_h, pt_h, ou_h, oi_h, op_h,
          idx_u, idx_i, idx_p, rw_u, rw_i, rw_p, sem):
        wid = lax.axis_index("s") * _NC + lax.axis_index("c")
        r0 = wid * _NCH
        pltpu.sync_copy(uid_h.at[pl.ds(r0, _NCH)], idx_u)
        pltpu.sync_copy(iid_h.at[pl.ds(r0, _NCH)], idx_i)
        pltpu.sync_copy(pid_h.at[pl.ds(r0, _NCH)], idx_p)
        copies = []
        for j in range(_NCH):
            copies.append(pltpu.async_copy(ut_h.at[idx_u.at[j]], rw_u.at[j], sem))
            copies.append(pltpu.async_copy(it_h.at[idx_i.at[j]], rw_i.at[j], sem))
            copies.append(pltpu.async_copy(pt_h.at[idx_p.at[j]], rw_p.at[j], sem))
        for c in copies:
            c.wait()
        pltpu.sync_copy(rw_u, ou_h.at[pl.ds(r0, _NCH)])
        pltpu.sync_copy(rw_i, oi_h.at[pl.ds(r0, _NCH)])
        pltpu.sync_copy(rw_p, op_h.at[pl.ds(r0, _NCH)])

    return k(uid2, iid2, pid2, user_table, item_table, pub_table)


_BT = 2048
_NB = B // _BT


def _tc_body(ue_r, ie_r, pe_r, la_r, eb_r, fm_r, de_r, av_r, pg_r,
             lt_r, et_r, ft_r, dt_r,
             wu1_r, bu1_r, wu2_r, bu2_r,
             wit_r, wil_r, wie_r, wif_r, wip_r, wid_r, wav_r, wpg_r,
             bi1_r, wi2_r, bi2_r, out_r):
    f32 = jnp.float32

    def mm(a, b):
        return jax.lax.dot_general(a, b, (((1,), (0,)), ((), ())),
                                   preferred_element_type=f32)

    def small_lookup(idx_col, n, table, wblock):
        cols = lax.broadcasted_iota(jnp.int32, (_BT, n), 1)
        oh = jnp.where(cols == idx_col, 1.0, 0.0).astype(f32)
        return mm(oh, mm(table, wblock))

    la = la_r[...]
    eb = eb_r[...]
    fm = fm_r[...]
    de = de_r[...]
    av = av_r[...]
    pg = pg_r[...]

    hidden_i = (mm(ie_r[...], wit_r[...])
                + mm(pe_r[...], wip_r[...])
                + small_lookup(la, 64, lt_r[...], wil_r[...])
                + small_lookup(eb, 8, et_r[...], wie_r[...])
                + small_lookup(fm, 16, ft_r[...], wif_r[...])
                + small_lookup(de, 24, dt_r[...], wid_r[...])
                + av * wav_r[...]
                + pg * wpg_r[...]
                + bi1_r[...])
    hi = hidden_i * jax.nn.sigmoid(hidden_i)
    item_o = mm(hi, wi2_r[...]) + bi2_r[...]

    hu_pre = mm(ue_r[...], wu1_r[...]) + bu1_r[...]
    hu = hu_pre * jax.nn.sigmoid(hu_pre)
    u = mm(hu, wu2_r[...]) + bu2_r[...]

    out_r[...] = jnp.sum(u * item_o, axis=1, keepdims=True)


def _tc_towers(ue, ie, pe, la, eb, fm, de, av, pg,
               lt, et, ft, dt,
               wu1, bu1, wu2, bu2,
               wit, wil, wie, wif, wip, wid, wav, wpg,
               bi1, wi2, bi2):
    bcol = pl.BlockSpec((_BT, 1), lambda i: (i, 0))
    bemb = pl.BlockSpec((_BT, D), lambda i: (i, 0))

    def full(x):
        return pl.BlockSpec(x.shape, lambda i: (0,) * x.ndim)

    in_specs = [bemb, bemb, bemb, bcol, bcol, bcol, bcol, bcol, bcol]
    in_specs += [full(x) for x in (lt, et, ft, dt,
                                   wu1, bu1, wu2, bu2,
                                   wit, wil, wie, wif, wip, wid, wav, wpg,
                                   bi1, wi2, bi2)]
    return pl.pallas_call(
        _tc_body,
        grid=(_NB,),
        in_specs=in_specs,
        out_specs=pl.BlockSpec((_BT, 1), lambda i: (i, 0)),
        out_shape=jax.ShapeDtypeStruct((B, 1), jnp.float32),
    )(ue, ie, pe, la, eb, fm, de, av, pg,
      lt, et, ft, dt,
      wu1, bu1, wu2, bu2,
      wit, wil, wie, wif, wip, wid, wav, wpg,
      bi1, wi2, bi2)


def kernel(user_id, item_id, language, is_ebook, format, publisher, pub_decade,
           avg_rating, num_pages,
           user_table, item_table, lang_table, ebook_table, format_table,
           pub_table, decade_table,
           W_u1, b_u1, W_u2, b_u2, W_i1, b_i1, W_i2, b_i2):
    f32 = jnp.float32
    uid2 = user_id.astype(jnp.int32).reshape(_NROW, _CH)
    iid2 = item_id.astype(jnp.int32).reshape(_NROW, _CH)
    pid2 = publisher.astype(jnp.int32).reshape(_NROW, _CH)

    ue, ie, pe = _sc_gather3(uid2, iid2, pid2, user_table, item_table, pub_table)
    ue = ue.reshape(B, D)
    ie = ie.reshape(B, D)
    pe = pe.reshape(B, D)

    la = language.astype(jnp.int32).reshape(B, 1)
    eb = is_ebook.astype(jnp.int32).reshape(B, 1)
    fm = format.astype(jnp.int32).reshape(B, 1)
    de = pub_decade.astype(jnp.int32).reshape(B, 1)
    av = avg_rating.astype(f32).reshape(B, 1)
    pg = num_pages.astype(f32).reshape(B, 1)

    et = jnp.zeros((8, D), f32).at[:2].set(ebook_table)
    dt = jnp.zeros((24, D), f32).at[:20].set(decade_table)

    wit = W_i1[0:32]
    wil = W_i1[32:64]
    wie = W_i1[64:96]
    wif = W_i1[96:128]
    wip = W_i1[128:160]
    wid = W_i1[160:192]
    wav = W_i1[192:193]
    wpg = W_i1[193:194]

    out = _tc_towers(ue, ie, pe, la, eb, fm, de, av, pg,
                     lang_table, et, format_table, dt,
                     W_u1, b_u1.reshape(1, D), W_u2, b_u2.reshape(1, D),
                     wit, wil, wie, wif, wip, wid, wav, wpg,
                     b_i1.reshape(1, D), W_i2, b_i2.reshape(1, D))
    return out.reshape(B)
